# baseline (speedup 1.0000x reference)
import functools

import jax
import jax.numpy as jnp
import numpy as np
from jax import lax
from jax.experimental import pallas as pl
from jax.experimental.pallas import tpu as pltpu
from jax.experimental.pallas import tpu_sc as plsc

N = 10000
E = 320000
R = 3
F_IN = 128
HID = 128
OUT = 64
B = 1024
EPS = 1e-5

NPAD = 10240
NC = 2
NS = 16
NW = NC * NS
LANES = 16
EW = E // NW
K = 80
NBLK = EW // K
ROWS_PER_TILE = NPAD // NS
BN_C = float(1.0 / np.sqrt(1.0 + EPS))

_mesh = plsc.VectorSubcoreMesh(core_axis_name="c", subcore_axis_name="s")
_sc_params = pltpu.CompilerParams(use_tc_tiling_on_sc=False)
_tc_params = pltpu.CompilerParams(vmem_limit_bytes=110 * 1024 * 1024)


def _worker_id():
    cid = lax.axis_index("c")
    sid = lax.axis_index("s")
    return cid, sid, cid * NS + sid


def _fill_rows(ref, nrows, ncols, value):
    nch = ncols // LANES

    def body(i, _):
        for j in range(nch):
            ref[i, pl.ds(j * LANES, LANES)] = jnp.full((LANES,), value,
                                                       jnp.float32)
        return 0

    lax.fori_loop(0, nrows, body, 0)


def _sc_deg_body(mr2_hbm, degp_hbm, ones_v, didx_v, buf_v, acc_sh):
    cid, sid, wid = _worker_id()
    _fill_rows(ones_v, K, LANES, 1.0)
    for r in range(R):
        _fill_rows(buf_v, 128, LANES, 0.0)
        for k in range(ROWS_PER_TILE // 128):
            pltpu.sync_copy(buf_v, acc_sh.at[pl.ds(sid * ROWS_PER_TILE
                                                   + k * 128, 128)])
        drow = (r * 2 * E + E + wid * EW) // K
        pltpu.sync_copy(mr2_hbm.at[pl.ds(drow, NBLK), :], didx_v)
        plsc.subcore_barrier()

        def eloop(i, _):
            pltpu.sync_copy(ones_v, acc_sh.at[didx_v.at[i]], add=True)
            return 0

        lax.fori_loop(0, NBLK, eloop, 0)
        plsc.subcore_barrier()

        for k in range(ROWS_PER_TILE // 128):
            off = sid * ROWS_PER_TILE + k * 128
            pltpu.sync_copy(acc_sh.at[pl.ds(off, 128)], buf_v)
            pltpu.sync_copy(buf_v, degp_hbm.at[r, cid, pl.ds(off, 128), :])


_sc_deg = pl.kernel(
    _sc_deg_body,
    out_type=jax.ShapeDtypeStruct((R, 2, NPAD, LANES), jnp.float32),
    mesh=_mesh,
    compiler_params=_sc_params,
    scratch_types=[
        pltpu.VMEM((K, LANES), jnp.float32),
        pltpu.VMEM((NBLK, K), jnp.int32),
        pltpu.VMEM((128, LANES), jnp.float32),
        pltpu.VMEM_SHARED((NPAD, LANES), jnp.float32),
    ],
)


def _make_sc_conv(D):
    def body(mr2_hbm, t0, t1, t2, sp_hbm, sidx_v, didx_v, rows0_v, rows1_v,
             buf_v, sem, acc_sh):
        cid, sid, wid = _worker_id()
        tables = (t0, t1, t2)
        for r in range(R):
            _fill_rows(buf_v, 32, D, 0.0)
            for k in range(ROWS_PER_TILE // 32):
                pltpu.sync_copy(buf_v, acc_sh.at[pl.ds(sid * ROWS_PER_TILE
                                                       + k * 32, 32)])
            srow = (r * 2 * E + wid * EW) // K
            pltpu.sync_copy(mr2_hbm.at[pl.ds(srow, NBLK), :], sidx_v)
            pltpu.sync_copy(mr2_hbm.at[pl.ds(srow + E // K, NBLK), :],
                            didx_v)
            plsc.subcore_barrier()

            table = tables[r]

            def fire(i, rows):
                pltpu.async_copy(table.at[sidx_v.at[i]], rows, sem)

            def drain(i, rows):
                pltpu.make_async_copy(table.at[sidx_v.at[i]], rows,
                                      sem).wait()

            def scatter(i, rows):
                pltpu.sync_copy(rows, acc_sh.at[didx_v.at[i]], add=True)

            fire(0, rows0_v)

            def eloop(i2, _):
                i = i2 * 2
                drain(i, rows0_v)
                fire(i + 1, rows1_v)
                scatter(i, rows0_v)
                drain(i + 1, rows1_v)
                fire(i + 2, rows0_v)
                scatter(i + 1, rows1_v)
                return 0

            lax.fori_loop(0, (NBLK - 1) // 2, eloop, 0)
            drain(NBLK - 1, rows0_v)
            scatter(NBLK - 1, rows0_v)
            plsc.subcore_barrier()

            for k in range(ROWS_PER_TILE // 32):
                off = sid * ROWS_PER_TILE + k * 32
                pltpu.sync_copy(acc_sh.at[pl.ds(off, 32)], buf_v)
                pltpu.sync_copy(buf_v, sp_hbm.at[r, cid, pl.ds(off, 32), :])

    return pl.kernel(
        body,
        out_type=jax.ShapeDtypeStruct((R, 2, NPAD, D), jnp.float32),
        mesh=_mesh,
        compiler_params=_sc_params,
        scratch_types=[
            pltpu.VMEM((NBLK, K), jnp.int32),
            pltpu.VMEM((NBLK, K), jnp.int32),
            pltpu.VMEM((K, D), jnp.float32),
            pltpu.VMEM((K, D), jnp.float32),
            pltpu.VMEM((32, D), jnp.float32),
            pltpu.SemaphoreType.DMA,
            pltpu.VMEM_SHARED((NPAD, D), jnp.float32),
        ],
    )


_sc_conv_hid = _make_sc_conv(HID)
_sc_conv_out = _make_sc_conv(OUT)


def _sc_gather_body(x0, x1, x2, bn_hbm, out_hbm, bidx_v, rows_v, sem):
    cid, sid, wid = _worker_id()
    per_w = B // NW
    boff = wid * per_w
    pltpu.sync_copy(bn_hbm.at[pl.ds(boff, per_w)], bidx_v)
    for r, tab in enumerate((x0, x1, x2)):
        pltpu.async_copy(tab.at[bidx_v], rows_v, sem).wait()
        pltpu.sync_copy(rows_v, out_hbm.at[pl.ds(boff, per_w),
                                           pl.ds(r * OUT, OUT)])


_sc_gather = pl.kernel(
    _sc_gather_body,
    out_type=jax.ShapeDtypeStruct((B, R * OUT), jnp.float32),
    mesh=_mesh,
    compiler_params=_sc_params,
    scratch_types=[
        pltpu.VMEM((B // NW,), jnp.int32),
        pltpu.VMEM((B // NW, OUT), jnp.float32),
        pltpu.SemaphoreType.DMA,
    ],
)


NB = 2000
NJ = N // NB


def _dinv_col(degp_ref):
    deg = degp_ref[0, 0, :, 0:1] + degp_ref[0, 1, :, 0:1] + 1.0
    return lax.rsqrt(deg)


def _tc1_body(f_ref, w1_ref, degp_ref, hh1_ref):
    dinv = _dinv_col(degp_ref)
    h = jnp.dot(f_ref[...], w1_ref[0],
                preferred_element_type=jnp.float32)
    hh1_ref[0] = h * dinv


def _tc1(features, W1, degp):
    return pl.pallas_call(
        _tc1_body,
        grid=(R, NJ),
        in_specs=[
            pl.BlockSpec((NB, F_IN), lambda r, j: (j, 0)),
            pl.BlockSpec((1, F_IN, HID), lambda r, j: (r, 0, 0)),
            pl.BlockSpec((1, 2, NB, LANES), lambda r, j: (r, 0, j, 0)),
        ],
        out_specs=pl.BlockSpec((1, NB, HID), lambda r, j: (r, j, 0)),
        out_shape=jax.ShapeDtypeStruct((R, N, HID), jnp.float32),
        compiler_params=_tc_params,
    )(features, W1, degp)


def _tc2_body(s1p_ref, hh1_ref, degp_ref, w2_ref, b1_ref, g1_ref, bt1_ref,
              hh2_ref):
    s = s1p_ref[0, 0] + s1p_ref[0, 1]
    dinv = _dinv_col(degp_ref)
    conv = dinv * (s + hh1_ref[0]) + b1_ref[0]
    x1 = jnp.maximum(conv * (BN_C) * g1_ref[0] + bt1_ref[0], 0.0)
    h2 = jnp.dot(x1, w2_ref[0], preferred_element_type=jnp.float32)
    hh2_ref[0] = h2 * dinv


def _tc2(s1p, hh1, degp, W2, b1, g1, beta1):
    return pl.pallas_call(
        _tc2_body,
        grid=(R, NJ),
        in_specs=[
            pl.BlockSpec((1, 2, NB, HID), lambda r, j: (r, 0, j, 0)),
            pl.BlockSpec((1, NB, HID), lambda r, j: (r, j, 0)),
            pl.BlockSpec((1, 2, NB, LANES), lambda r, j: (r, 0, j, 0)),
            pl.BlockSpec((1, HID, OUT), lambda r, j: (r, 0, 0)),
            pl.BlockSpec((1, 1, HID), lambda r, j: (r, 0, 0)),
            pl.BlockSpec((1, 1, HID), lambda r, j: (r, 0, 0)),
            pl.BlockSpec((1, 1, HID), lambda r, j: (r, 0, 0)),
        ],
        out_specs=pl.BlockSpec((1, NB, OUT), lambda r, j: (r, j, 0)),
        out_shape=jax.ShapeDtypeStruct((R, N, OUT), jnp.float32),
        compiler_params=_tc_params,
    )(s1p, hh1, degp, W2, b1[:, None, :], g1[:, None, :], beta1[:, None, :])


def _tc3_body(s2p_ref, hh2_ref, degp_ref, b2_ref, g2_ref, bt2_ref, x2_ref):
    s = s2p_ref[0, 0] + s2p_ref[0, 1]
    dinv = _dinv_col(degp_ref)
    conv = dinv * (s + hh2_ref[0]) + b2_ref[0]
    x2 = jnp.maximum(conv * (BN_C) * g2_ref[0] + bt2_ref[0], 0.0)
    m = jnp.max(x2, axis=1, keepdims=True)
    ex = jnp.exp(x2 - m)
    lse = jnp.log(jnp.sum(ex, axis=1, keepdims=True))
    x2_ref[0] = x2 - m - lse


def _tc3(s2p, hh2, degp, b2, g2, beta2):
    return pl.pallas_call(
        _tc3_body,
        grid=(R, NJ),
        in_specs=[
            pl.BlockSpec((1, 2, NB, OUT), lambda r, j: (r, 0, j, 0)),
            pl.BlockSpec((1, NB, OUT), lambda r, j: (r, j, 0)),
            pl.BlockSpec((1, 2, NB, LANES), lambda r, j: (r, 0, j, 0)),
            pl.BlockSpec((1, 1, OUT), lambda r, j: (r, 0, 0)),
            pl.BlockSpec((1, 1, OUT), lambda r, j: (r, 0, 0)),
            pl.BlockSpec((1, 1, OUT), lambda r, j: (r, 0, 0)),
        ],
        out_specs=pl.BlockSpec((1, NB, OUT), lambda r, j: (r, j, 0)),
        out_shape=jax.ShapeDtypeStruct((R, N, OUT), jnp.float32),
        compiler_params=_tc_params,
    )(s2p, hh2, degp, b2[:, None, :], g2[:, None, :], beta2[:, None, :])


@jax.jit
def _run(features, multi_r_data, batch_nodes, W1, b1, g1, beta1, W2, b2, g2,
         beta2):
    mr = multi_r_data.reshape(-1, K)
    degp = _sc_deg(mr)
    hh1 = _tc1(features, W1, degp)
    s1p = _sc_conv_hid(mr, hh1[0], hh1[1], hh1[2])
    hh2 = _tc2(s1p, hh1, degp, W2, b1, g1, beta1)
    s2p = _sc_conv_out(mr, hh2[0], hh2[1], hh2[2])
    x2 = _tc3(s2p, hh2, degp, b2, g2, beta2)
    return _sc_gather(x2[0], x2[1], x2[2], batch_nodes)


def kernel(features, multi_r_data, batch_nodes, W1, b1, g1, beta1, W2, b2,
           g2, beta2):
    return _run(features, multi_r_data, batch_nodes, W1, b1, g1, beta1, W2,
                b2, g2, beta2)


# speedup vs baseline: 1.2201x; 1.2201x over previous
import functools

import jax
import jax.numpy as jnp
import numpy as np
from jax import lax
from jax.experimental import pallas as pl
from jax.experimental.pallas import tpu as pltpu
from jax.experimental.pallas import tpu_sc as plsc

N = 10000
E = 320000
R = 3
F_IN = 128
HID = 128
OUT = 64
B = 1024
EPS = 1e-5

NPAD = 10240
NC = 2
NS = 16
NW = NC * NS
LANES = 16
EW = E // NW
K = 80
NBLK = EW // K
ROWS_PER_TILE = NPAD // NS
BN_C = float(1.0 / np.sqrt(1.0 + EPS))

_mesh = plsc.VectorSubcoreMesh(core_axis_name="c", subcore_axis_name="s")
_sc_params = pltpu.CompilerParams(use_tc_tiling_on_sc=False)
_tc_params = pltpu.CompilerParams(vmem_limit_bytes=110 * 1024 * 1024)


def _worker_id():
    cid = lax.axis_index("c")
    sid = lax.axis_index("s")
    return cid, sid, cid * NS + sid


def _fill_rows(ref, nrows, ncols, value):
    nch = ncols // LANES

    def body(i, _):
        for j in range(nch):
            ref[i, pl.ds(j * LANES, LANES)] = jnp.full((LANES,), value,
                                                       jnp.float32)
        return 0

    lax.fori_loop(0, nrows, body, 0)


def _sc_deg_body(mr2_hbm, degp_hbm, ones_v, didx_v, buf_v, acc_sh):
    cid, sid, wid = _worker_id()
    _fill_rows(ones_v, K, LANES, 1.0)
    for r in range(R):
        _fill_rows(buf_v, 128, LANES, 0.0)
        for k in range(ROWS_PER_TILE // 128):
            pltpu.sync_copy(buf_v, acc_sh.at[pl.ds(sid * ROWS_PER_TILE
                                                   + k * 128, 128)])
        drow = (r * 2 * E + E + wid * EW) // K
        pltpu.sync_copy(mr2_hbm.at[pl.ds(drow, NBLK), :], didx_v)
        plsc.subcore_barrier()

        def eloop(i, _):
            pltpu.sync_copy(ones_v, acc_sh.at[didx_v.at[i]], add=True)
            return 0

        lax.fori_loop(0, NBLK, eloop, 0)
        plsc.subcore_barrier()

        for k in range(ROWS_PER_TILE // 128):
            off = sid * ROWS_PER_TILE + k * 128
            pltpu.sync_copy(acc_sh.at[pl.ds(off, 128)], buf_v)
            pltpu.sync_copy(buf_v, degp_hbm.at[r, cid, pl.ds(off, 128), :])


_sc_deg = pl.kernel(
    _sc_deg_body,
    out_type=jax.ShapeDtypeStruct((R, 2, NPAD, LANES), jnp.float32),
    mesh=_mesh,
    compiler_params=_sc_params,
    scratch_types=[
        pltpu.VMEM((K, LANES), jnp.float32),
        pltpu.VMEM((NBLK, K), jnp.int32),
        pltpu.VMEM((128, LANES), jnp.float32),
        pltpu.VMEM_SHARED((NPAD, LANES), jnp.float32),
    ],
)


def _make_sc_conv(D):
    def body(mr2_hbm, t0, t1, t2, sp_hbm, sidx_v, didx_v, rows0_v, rows1_v,
             buf_v, sem, ssem, acc_sh):
        cid, sid, wid = _worker_id()
        tables = (t0, t1, t2)
        for r in range(R):
            _fill_rows(buf_v, 32, D, 0.0)
            for k in range(ROWS_PER_TILE // 32):
                pltpu.sync_copy(buf_v, acc_sh.at[pl.ds(sid * ROWS_PER_TILE
                                                       + k * 32, 32)])
            srow = (r * 2 * E + wid * EW) // K
            pltpu.sync_copy(mr2_hbm.at[pl.ds(srow, NBLK), :], sidx_v)
            pltpu.sync_copy(mr2_hbm.at[pl.ds(srow + E // K, NBLK), :],
                            didx_v)
            plsc.subcore_barrier()

            table = tables[r]

            def fire(i, rows):
                pltpu.async_copy(table.at[sidx_v.at[i]], rows, sem)

            def drain(i, rows):
                pltpu.make_async_copy(table.at[sidx_v.at[i]], rows,
                                      sem).wait()

            def sfire(i, rows):
                pltpu.async_copy(rows, acc_sh.at[didx_v.at[i]], ssem,
                                 add=True)

            def sdrain(i, rows):
                pltpu.make_async_copy(rows, acc_sh.at[didx_v.at[i]],
                                      ssem).wait()

            fire(0, rows0_v)
            fire(1, rows1_v)

            def eloop(i2, _):
                i = i2 * 2
                drain(i, rows0_v)
                sfire(i, rows0_v)
                drain(i + 1, rows1_v)
                sfire(i + 1, rows1_v)
                sdrain(i, rows0_v)

                @pl.when(i + 2 < NBLK)
                def _():
                    fire(i + 2, rows0_v)

                sdrain(i + 1, rows1_v)

                @pl.when(i + 3 < NBLK)
                def _():
                    fire(i + 3, rows1_v)

                return 0

            lax.fori_loop(0, (NBLK - 1) // 2, eloop, 0)
            drain(NBLK - 1, rows0_v)
            sfire(NBLK - 1, rows0_v)
            sdrain(NBLK - 1, rows0_v)
            plsc.subcore_barrier()

            for k in range(ROWS_PER_TILE // 32):
                off = sid * ROWS_PER_TILE + k * 32
                pltpu.sync_copy(acc_sh.at[pl.ds(off, 32)], buf_v)
                pltpu.sync_copy(buf_v, sp_hbm.at[r, cid, pl.ds(off, 32), :])

    return pl.kernel(
        body,
        out_type=jax.ShapeDtypeStruct((R, 2, NPAD, D), jnp.float32),
        mesh=_mesh,
        compiler_params=_sc_params,
        scratch_types=[
            pltpu.VMEM((NBLK, K), jnp.int32),
            pltpu.VMEM((NBLK, K), jnp.int32),
            pltpu.VMEM((K, D), jnp.float32),
            pltpu.VMEM((K, D), jnp.float32),
            pltpu.VMEM((32, D), jnp.float32),
            pltpu.SemaphoreType.DMA,
            pltpu.SemaphoreType.DMA,
            pltpu.VMEM_SHARED((NPAD, D), jnp.float32),
        ],
    )


_sc_conv_hid = _make_sc_conv(HID)
_sc_conv_out = _make_sc_conv(OUT)


def _sc_gather_body(x0, x1, x2, bn_hbm, out_hbm, bidx_v, rows_v, sem):
    cid, sid, wid = _worker_id()
    per_w = B // NW
    boff = wid * per_w
    pltpu.sync_copy(bn_hbm.at[pl.ds(boff, per_w)], bidx_v)
    for r, tab in enumerate((x0, x1, x2)):
        pltpu.async_copy(tab.at[bidx_v], rows_v, sem).wait()
        pltpu.sync_copy(rows_v, out_hbm.at[pl.ds(boff, per_w),
                                           pl.ds(r * OUT, OUT)])


_sc_gather = pl.kernel(
    _sc_gather_body,
    out_type=jax.ShapeDtypeStruct((B, R * OUT), jnp.float32),
    mesh=_mesh,
    compiler_params=_sc_params,
    scratch_types=[
        pltpu.VMEM((B // NW,), jnp.int32),
        pltpu.VMEM((B // NW, OUT), jnp.float32),
        pltpu.SemaphoreType.DMA,
    ],
)


NB = 2000
NJ = N // NB


def _dinv_col(degp_ref):
    deg = degp_ref[0, 0, :, 0:1] + degp_ref[0, 1, :, 0:1] + 1.0
    return lax.rsqrt(deg)


def _tc1_body(f_ref, w1_ref, degp_ref, hh1_ref):
    dinv = _dinv_col(degp_ref)
    h = jnp.dot(f_ref[...], w1_ref[0],
                preferred_element_type=jnp.float32)
    hh1_ref[0] = h * dinv


def _tc1(features, W1, degp):
    return pl.pallas_call(
        _tc1_body,
        grid=(R, NJ),
        in_specs=[
            pl.BlockSpec((NB, F_IN), lambda r, j: (j, 0)),
            pl.BlockSpec((1, F_IN, HID), lambda r, j: (r, 0, 0)),
            pl.BlockSpec((1, 2, NB, LANES), lambda r, j: (r, 0, j, 0)),
        ],
        out_specs=pl.BlockSpec((1, NB, HID), lambda r, j: (r, j, 0)),
        out_shape=jax.ShapeDtypeStruct((R, N, HID), jnp.float32),
        compiler_params=_tc_params,
    )(features, W1, degp)


def _tc2_body(s1p_ref, hh1_ref, degp_ref, w2_ref, b1_ref, g1_ref, bt1_ref,
              hh2_ref):
    s = s1p_ref[0, 0] + s1p_ref[0, 1]
    dinv = _dinv_col(degp_ref)
    conv = dinv * (s + hh1_ref[0]) + b1_ref[0]
    x1 = jnp.maximum(conv * (BN_C) * g1_ref[0] + bt1_ref[0], 0.0)
    h2 = jnp.dot(x1, w2_ref[0], preferred_element_type=jnp.float32)
    hh2_ref[0] = h2 * dinv


def _tc2(s1p, hh1, degp, W2, b1, g1, beta1):
    return pl.pallas_call(
        _tc2_body,
        grid=(R, NJ),
        in_specs=[
            pl.BlockSpec((1, 2, NB, HID), lambda r, j: (r, 0, j, 0)),
            pl.BlockSpec((1, NB, HID), lambda r, j: (r, j, 0)),
            pl.BlockSpec((1, 2, NB, LANES), lambda r, j: (r, 0, j, 0)),
            pl.BlockSpec((1, HID, OUT), lambda r, j: (r, 0, 0)),
            pl.BlockSpec((1, 1, HID), lambda r, j: (r, 0, 0)),
            pl.BlockSpec((1, 1, HID), lambda r, j: (r, 0, 0)),
            pl.BlockSpec((1, 1, HID), lambda r, j: (r, 0, 0)),
        ],
        out_specs=pl.BlockSpec((1, NB, OUT), lambda r, j: (r, j, 0)),
        out_shape=jax.ShapeDtypeStruct((R, N, OUT), jnp.float32),
        compiler_params=_tc_params,
    )(s1p, hh1, degp, W2, b1[:, None, :], g1[:, None, :], beta1[:, None, :])


def _tc3_body(s2p_ref, hh2_ref, degp_ref, b2_ref, g2_ref, bt2_ref, x2_ref):
    s = s2p_ref[0, 0] + s2p_ref[0, 1]
    dinv = _dinv_col(degp_ref)
    conv = dinv * (s + hh2_ref[0]) + b2_ref[0]
    x2 = jnp.maximum(conv * (BN_C) * g2_ref[0] + bt2_ref[0], 0.0)
    m = jnp.max(x2, axis=1, keepdims=True)
    ex = jnp.exp(x2 - m)
    lse = jnp.log(jnp.sum(ex, axis=1, keepdims=True))
    x2_ref[0] = x2 - m - lse


def _tc3(s2p, hh2, degp, b2, g2, beta2):
    return pl.pallas_call(
        _tc3_body,
        grid=(R, NJ),
        in_specs=[
            pl.BlockSpec((1, 2, NB, OUT), lambda r, j: (r, 0, j, 0)),
            pl.BlockSpec((1, NB, OUT), lambda r, j: (r, j, 0)),
            pl.BlockSpec((1, 2, NB, LANES), lambda r, j: (r, 0, j, 0)),
            pl.BlockSpec((1, 1, OUT), lambda r, j: (r, 0, 0)),
            pl.BlockSpec((1, 1, OUT), lambda r, j: (r, 0, 0)),
            pl.BlockSpec((1, 1, OUT), lambda r, j: (r, 0, 0)),
        ],
        out_specs=pl.BlockSpec((1, NB, OUT), lambda r, j: (r, j, 0)),
        out_shape=jax.ShapeDtypeStruct((R, N, OUT), jnp.float32),
        compiler_params=_tc_params,
    )(s2p, hh2, degp, b2[:, None, :], g2[:, None, :], beta2[:, None, :])


@jax.jit
def _run(features, multi_r_data, batch_nodes, W1, b1, g1, beta1, W2, b2, g2,
         beta2):
    mr = multi_r_data.reshape(-1, K)
    degp = _sc_deg(mr)
    hh1 = _tc1(features, W1, degp)
    s1p = _sc_conv_hid(mr, hh1[0], hh1[1], hh1[2])
    hh2 = _tc2(s1p, hh1, degp, W2, b1, g1, beta1)
    s2p = _sc_conv_out(mr, hh2[0], hh2[1], hh2[2])
    x2 = _tc3(s2p, hh2, degp, b2, g2, beta2)
    return _sc_gather(x2[0], x2[1], x2[2], batch_nodes)


def kernel(features, multi_r_data, batch_nodes, W1, b1, g1, beta1, W2, b2,
           g2, beta2):
    return _run(features, multi_r_data, batch_nodes, W1, b1, g1, beta1, W2,
                b2, g2, beta2)


# speedup vs baseline: 1.3730x; 1.1253x over previous
import jax
import jax.numpy as jnp
import numpy as np
from jax import lax
from jax.experimental import pallas as pl
from jax.experimental.pallas import tpu as pltpu
from jax.experimental.pallas import tpu_sc as plsc

N = 10000
E = 320000
R = 3
F_IN = 128
HID = 128
OUT = 64
B = 1024
EPS = 1e-5

NPAD = 10240
NC = 2
NS = 16
NW = NC * NS
LANES = 16
EW = E // NW
K = 80
NBLK = EW // K
RPT = NPAD // NS
BN_C = float(1.0 / np.sqrt(1.0 + EPS))

_mesh = plsc.VectorSubcoreMesh(core_axis_name="c", subcore_axis_name="s")
_sc_params = pltpu.CompilerParams(use_tc_tiling_on_sc=False)
_tc_params = pltpu.CompilerParams(vmem_limit_bytes=110 * 1024 * 1024)


def _worker_id():
    cid = lax.axis_index("c")
    sid = lax.axis_index("s")
    return cid, sid, cid * NS + sid


def _fill_rows(ref, nrows, ncols, value):
    nch = ncols // LANES

    def body(i, _):
        for j in range(nch):
            ref[i, pl.ds(j * LANES, LANES)] = jnp.full((LANES,), value,
                                                       jnp.float32)
        return 0

    lax.fori_loop(0, nrows, body, 0)


def _zero_acc(buf_v, acc_sh, sid, nb, D):
    _fill_rows(buf_v, nb, D, 0.0)
    for k in range(RPT // nb):
        pltpu.sync_copy(buf_v, acc_sh.at[pl.ds(sid * RPT + k * nb, nb)])


def _dump_acc(buf_v, acc_sh, out_hbm, cid, sid, nb):
    for k in range(RPT // nb):
        off = sid * RPT + k * nb
        pltpu.sync_copy(acc_sh.at[pl.ds(off, nb)], buf_v)
        pltpu.sync_copy(buf_v, out_hbm.at[cid, pl.ds(off, nb), :])


def _make_sc_deg(r):
    def body(mr2_hbm, degp_hbm, ones_v, didx_v, buf_v, acc_sh):
        cid, sid, wid = _worker_id()
        _fill_rows(ones_v, K, LANES, 1.0)
        _zero_acc(buf_v, acc_sh, sid, 128, LANES)
        drow = (r * 2 * E + E + wid * EW) // K
        pltpu.sync_copy(mr2_hbm.at[pl.ds(drow, NBLK), :], didx_v)
        plsc.subcore_barrier()

        def eloop(i, _):
            pltpu.sync_copy(ones_v, acc_sh.at[didx_v.at[i]], add=True)
            return 0

        lax.fori_loop(0, NBLK, eloop, 0)
        plsc.subcore_barrier()
        _dump_acc(buf_v, acc_sh, degp_hbm, cid, sid, 128)

    return pl.kernel(
        body,
        out_type=jax.ShapeDtypeStruct((2, NPAD, LANES), jnp.float32),
        mesh=_mesh,
        compiler_params=_sc_params,
        scratch_types=[
            pltpu.VMEM((K, LANES), jnp.float32),
            pltpu.VMEM((NBLK, K), jnp.int32),
            pltpu.VMEM((128, LANES), jnp.float32),
            pltpu.VMEM_SHARED((NPAD, LANES), jnp.float32),
        ],
    )


_sc_deg = [_make_sc_deg(r) for r in range(R)]


def _make_sc_conv(r, D):
    def body(mr2_hbm, table, sp_hbm, sidx_v, didx_v, rows0_v, rows1_v,
             buf_v, sem, ssem, acc_sh):
        cid, sid, wid = _worker_id()
        _zero_acc(buf_v, acc_sh, sid, 32, D)
        srow = (r * 2 * E + wid * EW) // K
        pltpu.sync_copy(mr2_hbm.at[pl.ds(srow, NBLK), :], sidx_v)
        pltpu.sync_copy(mr2_hbm.at[pl.ds(srow + E // K, NBLK), :], didx_v)
        plsc.subcore_barrier()

        def fire(i, rows):
            pltpu.async_copy(table.at[sidx_v.at[i]], rows, sem)

        def drain(i, rows):
            pltpu.make_async_copy(table.at[sidx_v.at[i]], rows, sem).wait()

        def sfire(i, rows):
            pltpu.async_copy(rows, acc_sh.at[didx_v.at[i]], ssem, add=True)

        def sdrain(i, rows):
            pltpu.make_async_copy(rows, acc_sh.at[didx_v.at[i]],
                                  ssem).wait()

        fire(0, rows0_v)
        fire(1, rows1_v)

        def eloop(i2, _):
            i = i2 * 2
            drain(i, rows0_v)
            sfire(i, rows0_v)
            drain(i + 1, rows1_v)
            sfire(i + 1, rows1_v)
            sdrain(i, rows0_v)

            @pl.when(i + 2 < NBLK)
            def _():
                fire(i + 2, rows0_v)

            sdrain(i + 1, rows1_v)

            @pl.when(i + 3 < NBLK)
            def _():
                fire(i + 3, rows1_v)

            return 0

        lax.fori_loop(0, (NBLK - 1) // 2, eloop, 0)
        drain(NBLK - 1, rows0_v)
        sfire(NBLK - 1, rows0_v)
        sdrain(NBLK - 1, rows0_v)
        plsc.subcore_barrier()
        _dump_acc(buf_v, acc_sh, sp_hbm, cid, sid, 32)

    return pl.kernel(
        body,
        out_type=jax.ShapeDtypeStruct((2, NPAD, D), jnp.float32),
        mesh=_mesh,
        compiler_params=_sc_params,
        scratch_types=[
            pltpu.VMEM((NBLK, K), jnp.int32),
            pltpu.VMEM((NBLK, K), jnp.int32),
            pltpu.VMEM((K, D), jnp.float32),
            pltpu.VMEM((K, D), jnp.float32),
            pltpu.VMEM((32, D), jnp.float32),
            pltpu.SemaphoreType.DMA,
            pltpu.SemaphoreType.DMA,
            pltpu.VMEM_SHARED((NPAD, D), jnp.float32),
        ],
    )


_sc_conv_hid = [_make_sc_conv(r, HID) for r in range(R)]
_sc_conv_out = [_make_sc_conv(r, OUT) for r in range(R)]


def _sc_gather_body(x0, x1, x2, bn_hbm, out_hbm, bidx_v, rows_v, sem):
    cid, sid, wid = _worker_id()
    per_w = B // NW
    boff = wid * per_w
    pltpu.sync_copy(bn_hbm.at[pl.ds(boff, per_w)], bidx_v)
    for r, tab in enumerate((x0, x1, x2)):
        pltpu.async_copy(tab.at[bidx_v], rows_v, sem).wait()
        pltpu.sync_copy(rows_v, out_hbm.at[pl.ds(boff, per_w),
                                           pl.ds(r * OUT, OUT)])


_sc_gather = pl.kernel(
    _sc_gather_body,
    out_type=jax.ShapeDtypeStruct((B, R * OUT), jnp.float32),
    mesh=_mesh,
    compiler_params=_sc_params,
    scratch_types=[
        pltpu.VMEM((B // NW,), jnp.int32),
        pltpu.VMEM((B // NW, OUT), jnp.float32),
        pltpu.SemaphoreType.DMA,
    ],
)


NB = 2000
NJ = N // NB


def _dinv_col(degp_ref):
    deg = degp_ref[0, :, 0:1] + degp_ref[1, :, 0:1] + 1.0
    return lax.rsqrt(deg)


def _tc1_body(f_ref, w1_ref, degp_ref, hh1_ref):
    dinv = _dinv_col(degp_ref)
    h = jnp.dot(f_ref[...], w1_ref[...],
                preferred_element_type=jnp.float32)
    hh1_ref[...] = h * dinv


def _tc1(features, w1, degp):
    return pl.pallas_call(
        _tc1_body,
        grid=(NJ,),
        in_specs=[
            pl.BlockSpec((NB, F_IN), lambda j: (j, 0)),
            pl.BlockSpec((F_IN, HID), lambda j: (0, 0)),
            pl.BlockSpec((2, NB, LANES), lambda j: (0, j, 0)),
        ],
        out_specs=pl.BlockSpec((NB, HID), lambda j: (j, 0)),
        out_shape=jax.ShapeDtypeStruct((N, HID), jnp.float32),
        compiler_params=_tc_params,
    )(features, w1, degp)


def _tc2_body(s1p_ref, hh1_ref, degp_ref, w2_ref, b1_ref, g1_ref, bt1_ref,
              hh2_ref):
    s = s1p_ref[0] + s1p_ref[1]
    dinv = _dinv_col(degp_ref)
    conv = dinv * (s + hh1_ref[...]) + b1_ref[...]
    x1 = jnp.maximum(conv * (BN_C) * g1_ref[...] + bt1_ref[...], 0.0)
    h2 = jnp.dot(x1, w2_ref[...], preferred_element_type=jnp.float32)
    hh2_ref[...] = h2 * dinv


def _tc2(s1p, hh1, degp, w2, b1, g1, beta1):
    return pl.pallas_call(
        _tc2_body,
        grid=(NJ,),
        in_specs=[
            pl.BlockSpec((2, NB, HID), lambda j: (0, j, 0)),
            pl.BlockSpec((NB, HID), lambda j: (j, 0)),
            pl.BlockSpec((2, NB, LANES), lambda j: (0, j, 0)),
            pl.BlockSpec((HID, OUT), lambda j: (0, 0)),
            pl.BlockSpec((1, HID), lambda j: (0, 0)),
            pl.BlockSpec((1, HID), lambda j: (0, 0)),
            pl.BlockSpec((1, HID), lambda j: (0, 0)),
        ],
        out_specs=pl.BlockSpec((NB, OUT), lambda j: (j, 0)),
        out_shape=jax.ShapeDtypeStruct((N, OUT), jnp.float32),
        compiler_params=_tc_params,
    )(s1p, hh1, degp, w2, b1[None, :], g1[None, :], beta1[None, :])


def _tc3_body(s2p_ref, hh2_ref, degp_ref, b2_ref, g2_ref, bt2_ref, x2_ref):
    s = s2p_ref[0] + s2p_ref[1]
    dinv = _dinv_col(degp_ref)
    conv = dinv * (s + hh2_ref[...]) + b2_ref[...]
    x2 = jnp.maximum(conv * (BN_C) * g2_ref[...] + bt2_ref[...], 0.0)
    m = jnp.max(x2, axis=1, keepdims=True)
    ex = jnp.exp(x2 - m)
    lse = jnp.log(jnp.sum(ex, axis=1, keepdims=True))
    x2_ref[...] = x2 - m - lse


def _tc3(s2p, hh2, degp, b2, g2, beta2):
    return pl.pallas_call(
        _tc3_body,
        grid=(NJ,),
        in_specs=[
            pl.BlockSpec((2, NB, OUT), lambda j: (0, j, 0)),
            pl.BlockSpec((NB, OUT), lambda j: (j, 0)),
            pl.BlockSpec((2, NB, LANES), lambda j: (0, j, 0)),
            pl.BlockSpec((1, OUT), lambda j: (0, 0)),
            pl.BlockSpec((1, OUT), lambda j: (0, 0)),
            pl.BlockSpec((1, OUT), lambda j: (0, 0)),
        ],
        out_specs=pl.BlockSpec((NB, OUT), lambda j: (j, 0)),
        out_shape=jax.ShapeDtypeStruct((N, OUT), jnp.float32),
        compiler_params=_tc_params,
    )(s2p, hh2, degp, b2[None, :], g2[None, :], beta2[None, :])


@jax.jit
def _run(features, multi_r_data, batch_nodes, W1, b1, g1, beta1, W2, b2, g2,
         beta2):
    mr = multi_r_data.reshape(-1, K)
    degp = [_sc_deg[r](mr) for r in range(R)]
    x2 = []
    for r in range(R):
        hh1 = _tc1(features, W1[r], degp[r])
        s1p = _sc_conv_hid[r](mr, hh1)
        hh2 = _tc2(s1p, hh1, degp[r], W2[r], b1[r], g1[r], beta1[r])
        s2p = _sc_conv_out[r](mr, hh2)
        x2.append(_tc3(s2p, hh2, degp[r], b2[r], g2[r], beta2[r]))
    return _sc_gather(x2[0], x2[1], x2[2], batch_nodes)


def kernel(features, multi_r_data, batch_nodes, W1, b1, g1, beta1, W2, b2,
           g2, beta2):
    return _run(features, multi_r_data, batch_nodes, W1, b1, g1, beta1, W2,
                b2, g2, beta2)
